# baseline (speedup 1.0000x reference)
import functools

import jax
import jax.numpy as jnp
from jax import lax
from jax.experimental import pallas as pl
from jax.experimental.pallas import tpu as pltpu
from jax.experimental.pallas import tpu_sc as plsc

_M, _HX, _HY, _LXY = 8, 512, 512, 8
_U, _V, _LUV = 400, 400, 8
_N = 1048576
_LO = _LXY + _LUV

_NC, _NS, _L = 2, 16, 16
_NW = _NC * _NS
_PW = _N // _NW
_C = 512
_NCH = _PW // _C
_NG = _C // _L
_RB = 4 * _C
_IBLK = 128
_NBLK = _RB // _IBLK


def _sc_body(m_hbm, h_hbm, u_hbm, v_hbm, fxy_hbm, fuv_hbm,
             out_hbm,
             m_v, h_v, u_v, v_v, idxxy_v, idxuv_v, wxy_v, wuv_v,
             bufxy_v, bufuv_v, out_v, sem_in, sem_xy, sem_uv, sem_out):
    wid = lax.axis_index("s") * _NC + lax.axis_index("c")
    base_w = wid * _PW
    iota = lax.iota(jnp.int32, _L)

    def corners(find, size):
        find = jnp.where(find >= float(size), jnp.full((_L,), float(size - 1)),
                         find)
        i1 = find.astype(jnp.int32)
        fr = find - i1.astype(jnp.float32)
        i2 = i1 + 1
        i2 = jnp.where(i2 >= size, jnp.zeros((_L,), jnp.int32), i2)
        return i1, i2, fr

    @pl.loop(0, _NCH)
    def chunk(k):
        cb = base_w + k * _C

        cm = pltpu.async_copy(m_hbm.at[pl.ds(cb, _C)], m_v, sem_in)
        ch = pltpu.async_copy(h_hbm.at[pl.ds(cb, _C)], h_v, sem_in)
        cu = pltpu.async_copy(u_hbm.at[pl.ds(cb, _C)], u_v, sem_in)
        cv = pltpu.async_copy(v_hbm.at[pl.ds(cb, _C)], v_v, sem_in)
        cm.wait(); ch.wait(); cu.wait(); cv.wait()

        @pl.loop(0, _NG)
        def grp(gi):
            off = gi * _L
            pts = off + iota
            mv = m_v[pl.ds(off, _L)]
            hx = plsc.load_gather(h_v, [pts, jnp.zeros((_L,), jnp.int32)])
            hy = plsc.load_gather(h_v, [pts, jnp.ones((_L,), jnp.int32)])
            uu = u_v[pl.ds(off, _L)]
            vv = v_v[pl.ds(off, _L)]

            i1, i2, ir = corners((hx + 1.0) * (0.5 * _HX), _HX)
            j1, j2, jr = corners((hy + 1.0) * (0.5 * _HY), _HY)
            base = mv * (_HX * _HY)
            a1 = base + i1 * _HY
            a2 = base + i2 * _HY
            idxxy_v[pl.ds(0 * _C + off, _L)] = a1 + j1
            idxxy_v[pl.ds(1 * _C + off, _L)] = a2 + j1
            idxxy_v[pl.ds(2 * _C + off, _L)] = a1 + j2
            idxxy_v[pl.ds(3 * _C + off, _L)] = a2 + j2
            omi = 1.0 - ir
            omj = 1.0 - jr
            wxy_v[pl.ds(0 * _C + off, _L)] = omi * omj
            wxy_v[pl.ds(1 * _C + off, _L)] = ir * omj
            wxy_v[pl.ds(2 * _C + off, _L)] = omi * jr
            wxy_v[pl.ds(3 * _C + off, _L)] = ir * jr

            p1, p2, pr = corners(uu * float(_U), _U)
            q1, q2, qr = corners(vv * float(_V), _V)
            baseu = mv * (_U * _V)
            b1 = baseu + p1 * _V
            b2 = baseu + p2 * _V
            idxuv_v[pl.ds(0 * _C + off, _L)] = b1 + q1
            idxuv_v[pl.ds(1 * _C + off, _L)] = b2 + q1
            idxuv_v[pl.ds(2 * _C + off, _L)] = b1 + q2
            idxuv_v[pl.ds(3 * _C + off, _L)] = b2 + q2
            omp = 1.0 - pr
            omq = 1.0 - qr
            wuv_v[pl.ds(0 * _C + off, _L)] = omp * omq
            wuv_v[pl.ds(1 * _C + off, _L)] = pr * omq
            wuv_v[pl.ds(2 * _C + off, _L)] = omp * qr
            wuv_v[pl.ds(3 * _C + off, _L)] = pr * qr

        @pl.loop(0, _NBLK)
        def fire(b):
            o = b * _IBLK
            pltpu.async_copy(fxy_hbm.at[idxxy_v.at[pl.ds(o, _IBLK)]],
                             bufxy_v.at[pl.ds(o, _IBLK)], sem_xy)
            pltpu.async_copy(fuv_hbm.at[idxuv_v.at[pl.ds(o, _IBLK)]],
                             bufuv_v.at[pl.ds(o, _IBLK)], sem_uv)

        @pl.when(k > 0)
        def _():
            pltpu.make_async_copy(out_v, out_hbm.at[pl.ds(0, _C * _LO // 128)],
                                  sem_out).wait()

        pltpu.make_async_copy(fxy_hbm.at[pl.ds(0, _RB)], bufxy_v,
                              sem_xy).wait()
        pltpu.make_async_copy(fuv_hbm.at[pl.ds(0, _RB)], bufuv_v,
                              sem_uv).wait()

        @pl.loop(0, _NG)
        def blend(gi):
            off = gi * _L
            pts = off + iota
            for (buf, wv, cbase) in ((bufxy_v, wxy_v, 0),
                                     (bufuv_v, wuv_v, _LXY)):
                w11 = wv[pl.ds(0 * _C + off, _L)]
                w21 = wv[pl.ds(1 * _C + off, _L)]
                w12 = wv[pl.ds(2 * _C + off, _L)]
                w22 = wv[pl.ds(3 * _C + off, _L)]
                r11 = pts
                r21 = pts + 1 * _C
                r12 = pts + 2 * _C
                r22 = pts + 3 * _C
                for l in range(_LXY):
                    col = jnp.full((_L,), l, jnp.int32)
                    g11 = plsc.load_gather(buf, [r11, col])
                    g21 = plsc.load_gather(buf, [r21, col])
                    g12 = plsc.load_gather(buf, [r12, col])
                    g22 = plsc.load_gather(buf, [r22, col])
                    acc = g11 * w11 + g21 * w21 + g12 * w12 + g22 * w22
                    plsc.store_scatter(
                        out_v,
                        [(iota >> 3) + (off >> 3),
                         (iota & 7) * _LO + (cbase + l)], acc)

        orow0 = pl.multiple_of((base_w + k * _C) * _LO // 128, _C * _LO // 128)
        pltpu.async_copy(out_v, out_hbm.at[pl.ds(orow0, _C * _LO // 128)],
                         sem_out)

    pltpu.make_async_copy(out_v, out_hbm.at[pl.ds(0, _C * _LO // 128)],
                          sem_out).wait()


_sc_kernel = pl.kernel(
    _sc_body,
    out_type=jax.ShapeDtypeStruct((_N * _LO // 128, 128), jnp.float32),
    mesh=plsc.VectorSubcoreMesh(core_axis_name="c", subcore_axis_name="s"),
    compiler_params=pltpu.CompilerParams(needs_layout_passes=False,
                                         use_tc_tiling_on_sc=False),
    scratch_types=[
        pltpu.VMEM((_C,), jnp.int32),
        pltpu.VMEM((_C, 2), jnp.float32),
        pltpu.VMEM((_C,), jnp.float32),
        pltpu.VMEM((_C,), jnp.float32),
        pltpu.VMEM((_RB,), jnp.int32),
        pltpu.VMEM((_RB,), jnp.int32),
        pltpu.VMEM((_RB,), jnp.float32),
        pltpu.VMEM((_RB,), jnp.float32),
        pltpu.VMEM((_RB, _LXY), jnp.float32),
        pltpu.VMEM((_RB, _LUV), jnp.float32),
        pltpu.VMEM((_C * _LO // 128, 128), jnp.float32),
        pltpu.SemaphoreType.DMA,
        pltpu.SemaphoreType.DMA,
        pltpu.SemaphoreType.DMA,
        pltpu.SemaphoreType.DMA,
    ],
)


@jax.jit
def kernel(m, h, u, v, Fxy, Fuv):
    fxy = Fxy.reshape(_M * _HX * _HY, _LXY)
    fuv = Fuv.reshape(_M * _U * _V, _LUV)
    out = _sc_kernel(m, h, u, v, fxy, fuv)
    return out.reshape(_N, _LO)


# speedup vs baseline: 1.0984x; 1.0984x over previous
import functools

import jax
import jax.numpy as jnp
from jax import lax
from jax.experimental import pallas as pl
from jax.experimental.pallas import tpu as pltpu
from jax.experimental.pallas import tpu_sc as plsc

_M, _HX, _HY, _LXY = 8, 512, 512, 8
_U, _V, _LUV = 400, 400, 8
_N = 1048576
_LO = _LXY + _LUV

_NC, _NS, _L = 2, 16, 16
_NW = _NC * _NS
_PW = _N // _NW
_C = 512
_NCH = _PW // _C
_NG = _C // _L
_RB = 4 * _C
_IBLK = 128
_NBLK = _RB // _IBLK


def _sc_body(m_hbm, h_hbm, u_hbm, v_hbm, fxy_hbm, fuv_hbm,
             out_hbm,
             m_v, h_v, u_v, v_v, idxxy_v, idxuv_v, wxy_v, wuv_v,
             bufxy_v, bufuv_v, out_v, sem_in, sem_xy, sem_uv, sem_out):
    wid = lax.axis_index("s") * _NC + lax.axis_index("c")
    base_w = wid * _PW
    iota = lax.iota(jnp.int32, _L)

    def corners(find, size):
        find = jnp.where(find >= float(size), jnp.full((_L,), float(size - 1)),
                         find)
        i1 = find.astype(jnp.int32)
        fr = find - i1.astype(jnp.float32)
        i2 = i1 + 1
        i2 = jnp.where(i2 >= size, jnp.zeros((_L,), jnp.int32), i2)
        return i1, i2, fr

    @pl.loop(0, _NCH)
    def chunk(k):
        cb = base_w + k * _C

        cm = pltpu.async_copy(m_hbm.at[pl.ds(cb, _C)], m_v, sem_in)
        ch = pltpu.async_copy(h_hbm.at[pl.ds(cb * 2, _C * 2)], h_v, sem_in)
        cu = pltpu.async_copy(u_hbm.at[pl.ds(cb, _C)], u_v, sem_in)
        cv = pltpu.async_copy(v_hbm.at[pl.ds(cb, _C)], v_v, sem_in)
        cm.wait(); ch.wait(); cu.wait(); cv.wait()

        @pl.loop(0, _NG)
        def grp(gi):
            off = gi * _L
            pts = off + iota
            mv = m_v[pl.ds(off, _L)]
            hx = plsc.load_gather(h_v, [pts * 2])
            hy = plsc.load_gather(h_v, [pts * 2 + 1])
            uu = u_v[pl.ds(off, _L)]
            vv = v_v[pl.ds(off, _L)]

            i1, i2, ir = corners((hx + 1.0) * (0.5 * _HX), _HX)
            j1, j2, jr = corners((hy + 1.0) * (0.5 * _HY), _HY)
            base = mv * (_HX * _HY)
            a1 = base + i1 * _HY
            a2 = base + i2 * _HY
            idxxy_v[pl.ds(0 * _C + off, _L)] = a1 + j1
            idxxy_v[pl.ds(1 * _C + off, _L)] = a2 + j1
            idxxy_v[pl.ds(2 * _C + off, _L)] = a1 + j2
            idxxy_v[pl.ds(3 * _C + off, _L)] = a2 + j2
            omi = 1.0 - ir
            omj = 1.0 - jr
            wxy_v[pl.ds(0 * _C + off, _L)] = omi * omj
            wxy_v[pl.ds(1 * _C + off, _L)] = ir * omj
            wxy_v[pl.ds(2 * _C + off, _L)] = omi * jr
            wxy_v[pl.ds(3 * _C + off, _L)] = ir * jr

            p1, p2, pr = corners(uu * float(_U), _U)
            q1, q2, qr = corners(vv * float(_V), _V)
            baseu = mv * (_U * _V)
            b1 = baseu + p1 * _V
            b2 = baseu + p2 * _V
            idxuv_v[pl.ds(0 * _C + off, _L)] = b1 + q1
            idxuv_v[pl.ds(1 * _C + off, _L)] = b2 + q1
            idxuv_v[pl.ds(2 * _C + off, _L)] = b1 + q2
            idxuv_v[pl.ds(3 * _C + off, _L)] = b2 + q2
            omp = 1.0 - pr
            omq = 1.0 - qr
            wuv_v[pl.ds(0 * _C + off, _L)] = omp * omq
            wuv_v[pl.ds(1 * _C + off, _L)] = pr * omq
            wuv_v[pl.ds(2 * _C + off, _L)] = omp * qr
            wuv_v[pl.ds(3 * _C + off, _L)] = pr * qr

        @pl.loop(0, _NBLK)
        def fire(b):
            o = b * _IBLK
            pltpu.async_copy(fxy_hbm.at[idxxy_v.at[pl.ds(o, _IBLK)]],
                             bufxy_v.at[pl.ds(o, _IBLK)], sem_xy)
            pltpu.async_copy(fuv_hbm.at[idxuv_v.at[pl.ds(o, _IBLK)]],
                             bufuv_v.at[pl.ds(o, _IBLK)], sem_uv)

        @pl.when(k > 0)
        def _():
            pltpu.make_async_copy(out_v, out_hbm.at[pl.ds(0, _C * _LO)],
                                  sem_out).wait()

        pltpu.make_async_copy(fxy_hbm.at[pl.ds(0, _RB)], bufxy_v,
                              sem_xy).wait()
        pltpu.make_async_copy(fuv_hbm.at[pl.ds(0, _RB)], bufuv_v,
                              sem_uv).wait()

        @pl.loop(0, _NG)
        def blend(gi):
            off = gi * _L
            pts = off + iota
            for (buf, wv, cbase) in ((bufxy_v, wxy_v, 0),
                                     (bufuv_v, wuv_v, _LXY)):
                w11 = wv[pl.ds(0 * _C + off, _L)]
                w21 = wv[pl.ds(1 * _C + off, _L)]
                w12 = wv[pl.ds(2 * _C + off, _L)]
                w22 = wv[pl.ds(3 * _C + off, _L)]
                r11 = pts
                r21 = pts + 1 * _C
                r12 = pts + 2 * _C
                r22 = pts + 3 * _C
                for l in range(_LXY):
                    col = jnp.full((_L,), l, jnp.int32)
                    g11 = plsc.load_gather(buf, [r11, col])
                    g21 = plsc.load_gather(buf, [r21, col])
                    g12 = plsc.load_gather(buf, [r12, col])
                    g22 = plsc.load_gather(buf, [r22, col])
                    acc = g11 * w11 + g21 * w21 + g12 * w12 + g22 * w22
                    plsc.store_scatter(out_v, [pts * _LO + (cbase + l)], acc)

        o0 = pl.multiple_of((base_w + k * _C) * _LO, _C * _LO)
        pltpu.async_copy(out_v, out_hbm.at[pl.ds(o0, _C * _LO)], sem_out)

    pltpu.make_async_copy(out_v, out_hbm.at[pl.ds(0, _C * _LO)],
                          sem_out).wait()


_sc_kernel = pl.kernel(
    _sc_body,
    out_type=jax.ShapeDtypeStruct((_N * _LO,), jnp.float32),
    mesh=plsc.VectorSubcoreMesh(core_axis_name="c", subcore_axis_name="s"),
    compiler_params=pltpu.CompilerParams(needs_layout_passes=False,
                                         use_tc_tiling_on_sc=False),
    scratch_types=[
        pltpu.VMEM((_C,), jnp.int32),
        pltpu.VMEM((_C * 2,), jnp.float32),
        pltpu.VMEM((_C,), jnp.float32),
        pltpu.VMEM((_C,), jnp.float32),
        pltpu.VMEM((_RB,), jnp.int32),
        pltpu.VMEM((_RB,), jnp.int32),
        pltpu.VMEM((_RB,), jnp.float32),
        pltpu.VMEM((_RB,), jnp.float32),
        pltpu.VMEM((_RB, _LXY), jnp.float32),
        pltpu.VMEM((_RB, _LUV), jnp.float32),
        pltpu.VMEM((_C * _LO,), jnp.float32),
        pltpu.SemaphoreType.DMA,
        pltpu.SemaphoreType.DMA,
        pltpu.SemaphoreType.DMA,
        pltpu.SemaphoreType.DMA,
    ],
)


@jax.jit
def kernel(m, h, u, v, Fxy, Fuv):
    fxy = Fxy.reshape(_M * _HX * _HY, _LXY)
    fuv = Fuv.reshape(_M * _U * _V, _LUV)
    out = _sc_kernel(m, h.reshape(_N * 2), u, v, fxy, fuv)
    return out.reshape(_N, _LO)


# speedup vs baseline: 1.9740x; 1.7971x over previous
import functools

import jax
import jax.numpy as jnp
from jax import lax
from jax.experimental import pallas as pl
from jax.experimental.pallas import tpu as pltpu
from jax.experimental.pallas import tpu_sc as plsc

_M, _HX, _HY, _LXY = 8, 512, 512, 8
_U, _V, _LUV = 400, 400, 8
_N = 1048576
_LO = _LXY + _LUV

_NC, _NS, _L = 2, 16, 16
_NW = _NC * _NS
_PW = _N // _NW
_C = 512
_NCH = _PW // _C
_NG = _C // _L
_RB = 4 * _C
_IBLK = 128
_NBLK = _RB // _IBLK


def _sc_body(m_hbm, h_hbm, u_hbm, v_hbm, fxy_hbm, fuv_hbm,
             out_hbm,
             m_v, h_v, u_v, v_v, idxxy_v, idxuv_v, wxy_v, wuv_v,
             bufxy_v, bufuv_v, out_v, sem_in, sem_xy, sem_uv, sem_out):
    wid = lax.axis_index("s") * _NC + lax.axis_index("c")
    base_w = wid * _PW
    iota = lax.iota(jnp.int32, _L)

    def corners(find, size):
        find = jnp.where(find >= float(size), jnp.full((_L,), float(size - 1)),
                         find)
        i1 = find.astype(jnp.int32)
        fr = find - i1.astype(jnp.float32)
        i2 = i1 + 1
        i2 = jnp.where(i2 >= size, jnp.zeros((_L,), jnp.int32), i2)
        return i1, i2, fr

    @pl.loop(0, _NCH)
    def chunk(k):
        cb = base_w + k * _C

        cm = pltpu.async_copy(m_hbm.at[pl.ds(cb, _C)], m_v, sem_in)
        blk0 = pl.multiple_of(cb // 128, _C // 128)
        ch = pltpu.async_copy(h_hbm.at[pl.ds(blk0, _C // 128)], h_v, sem_in)
        cu = pltpu.async_copy(u_hbm.at[pl.ds(cb, _C)], u_v, sem_in)
        cv = pltpu.async_copy(v_hbm.at[pl.ds(cb, _C)], v_v, sem_in)
        cm.wait(); ch.wait(); cu.wait(); cv.wait()

        @pl.loop(0, _NG)
        def grp(gi):
            off = gi * _L
            pts = off + iota
            mv = m_v[pl.ds(off, _L)]
            hx = h_v[gi >> 3, 0, pl.ds((gi & 7) * _L, _L)]
            hy = h_v[gi >> 3, 1, pl.ds((gi & 7) * _L, _L)]
            uu = u_v[pl.ds(off, _L)]
            vv = v_v[pl.ds(off, _L)]

            i1, i2, ir = corners((hx + 1.0) * (0.5 * _HX), _HX)
            j1, j2, jr = corners((hy + 1.0) * (0.5 * _HY), _HY)
            base = mv * (_HX * _HY)
            a1 = base + i1 * _HY
            a2 = base + i2 * _HY
            idxxy_v[pl.ds(0 * _C + off, _L)] = a1 + j1
            idxxy_v[pl.ds(1 * _C + off, _L)] = a2 + j1
            idxxy_v[pl.ds(2 * _C + off, _L)] = a1 + j2
            idxxy_v[pl.ds(3 * _C + off, _L)] = a2 + j2
            omi = 1.0 - ir
            omj = 1.0 - jr
            wxy_v[pl.ds(0 * _C + off, _L)] = omi * omj
            wxy_v[pl.ds(1 * _C + off, _L)] = ir * omj
            wxy_v[pl.ds(2 * _C + off, _L)] = omi * jr
            wxy_v[pl.ds(3 * _C + off, _L)] = ir * jr

            p1, p2, pr = corners(uu * float(_U), _U)
            q1, q2, qr = corners(vv * float(_V), _V)
            baseu = mv * (_U * _V)
            b1 = baseu + p1 * _V
            b2 = baseu + p2 * _V
            idxuv_v[pl.ds(0 * _C + off, _L)] = b1 + q1
            idxuv_v[pl.ds(1 * _C + off, _L)] = b2 + q1
            idxuv_v[pl.ds(2 * _C + off, _L)] = b1 + q2
            idxuv_v[pl.ds(3 * _C + off, _L)] = b2 + q2
            omp = 1.0 - pr
            omq = 1.0 - qr
            wuv_v[pl.ds(0 * _C + off, _L)] = omp * omq
            wuv_v[pl.ds(1 * _C + off, _L)] = pr * omq
            wuv_v[pl.ds(2 * _C + off, _L)] = omp * qr
            wuv_v[pl.ds(3 * _C + off, _L)] = pr * qr

        @pl.loop(0, _NBLK)
        def fire(b):
            o = b * _IBLK
            pltpu.async_copy(fxy_hbm.at[idxxy_v.at[pl.ds(o, _IBLK)]],
                             bufxy_v.at[pl.ds(o, _IBLK)], sem_xy)
            pltpu.async_copy(fuv_hbm.at[idxuv_v.at[pl.ds(o, _IBLK)]],
                             bufuv_v.at[pl.ds(o, _IBLK)], sem_uv)

        @pl.when(k > 0)
        def _():
            pltpu.make_async_copy(out_v, out_hbm.at[pl.ds(0, _C // 128)],
                                  sem_out).wait()

        pltpu.make_async_copy(fxy_hbm.at[pl.ds(0, _RB)], bufxy_v,
                              sem_xy).wait()
        pltpu.make_async_copy(fuv_hbm.at[pl.ds(0, _RB)], bufuv_v,
                              sem_uv).wait()

        @pl.loop(0, _NG)
        def blend(gi):
            off = gi * _L
            pts = off + iota
            for (buf, wv, cbase) in ((bufxy_v, wxy_v, 0),
                                     (bufuv_v, wuv_v, _LXY)):
                w11 = wv[pl.ds(0 * _C + off, _L)]
                w21 = wv[pl.ds(1 * _C + off, _L)]
                w12 = wv[pl.ds(2 * _C + off, _L)]
                w22 = wv[pl.ds(3 * _C + off, _L)]
                r11 = pts
                r21 = pts + 1 * _C
                r12 = pts + 2 * _C
                r22 = pts + 3 * _C
                for l in range(_LXY):
                    col = jnp.full((_L,), l, jnp.int32)
                    g11 = plsc.load_gather(buf, [r11, col])
                    g21 = plsc.load_gather(buf, [r21, col])
                    g12 = plsc.load_gather(buf, [r12, col])
                    g22 = plsc.load_gather(buf, [r22, col])
                    acc = g11 * w11 + g21 * w21 + g12 * w12 + g22 * w22
                    out_v[gi >> 3, cbase + l, pl.ds((gi & 7) * _L, _L)] = acc

        o0 = pl.multiple_of((base_w + k * _C) // 128, _C // 128)
        pltpu.async_copy(out_v, out_hbm.at[pl.ds(o0, _C // 128)], sem_out)

    pltpu.make_async_copy(out_v, out_hbm.at[pl.ds(0, _C // 128)],
                          sem_out).wait()


_sc_kernel = pl.kernel(
    _sc_body,
    out_type=jax.ShapeDtypeStruct((_N // 128, _LO, 128), jnp.float32),
    mesh=plsc.VectorSubcoreMesh(core_axis_name="c", subcore_axis_name="s"),
    compiler_params=pltpu.CompilerParams(needs_layout_passes=False,
                                         use_tc_tiling_on_sc=False),
    scratch_types=[
        pltpu.VMEM((_C,), jnp.int32),
        pltpu.VMEM((_C // 128, 2, 128), jnp.float32),
        pltpu.VMEM((_C,), jnp.float32),
        pltpu.VMEM((_C,), jnp.float32),
        pltpu.VMEM((_RB,), jnp.int32),
        pltpu.VMEM((_RB,), jnp.int32),
        pltpu.VMEM((_RB,), jnp.float32),
        pltpu.VMEM((_RB,), jnp.float32),
        pltpu.VMEM((_RB, _LXY), jnp.float32),
        pltpu.VMEM((_RB, _LUV), jnp.float32),
        pltpu.VMEM((_C // 128, _LO, 128), jnp.float32),
        pltpu.SemaphoreType.DMA,
        pltpu.SemaphoreType.DMA,
        pltpu.SemaphoreType.DMA,
        pltpu.SemaphoreType.DMA,
    ],
)


@jax.jit
def kernel(m, h, u, v, Fxy, Fuv):
    fxy = Fxy.reshape(_M * _HX * _HY, _LXY)
    fuv = Fuv.reshape(_M * _U * _V, _LUV)
    h3 = h.reshape(_N // 128, 128, 2).transpose(0, 2, 1)
    out = _sc_kernel(m, h3, u, v, fxy, fuv)
    return out.transpose(0, 2, 1).reshape(_N, _LO)


# speedup vs baseline: 2.2824x; 1.1563x over previous
import jax
import jax.numpy as jnp
from jax import lax
from jax.experimental import pallas as pl
from jax.experimental.pallas import tpu as pltpu
from jax.experimental.pallas import tpu_sc as plsc

_M, _HX, _HY, _LXY = 8, 512, 512, 8
_U, _V, _LUV = 400, 400, 8
_N = 1048576

_NC, _NS, _L = 2, 16, 16
_NW = _NC * _NS
_PW = _N // _NW
_C = 512
_NCH = _PW // _C
_NG = _C // _L
_RB = 4 * _C
_IBLK = 128
_NBLK = _RB // _IBLK


def _corners(find, size):
    find = jnp.where(find >= float(size), jnp.full((_L,), float(size - 1)),
                     find)
    i1 = find.astype(jnp.int32)
    fr = find - i1.astype(jnp.float32)
    i2 = i1 + 1
    i2 = jnp.where(i2 >= size, jnp.zeros((_L,), jnp.int32), i2)
    return i1, i2, fr


def _plane_kernel(I, J, LD, use_h):
    def body(*refs):
        if use_h:
            (m_hbm, h_hbm, f_hbm, out_hbm,
             m_v, h_v, idx_v, w_v, buf_v, out_v,
             sem_in, sem_g, sem_out) = refs
        else:
            (m_hbm, u_hbm, v_hbm, f_hbm, out_hbm,
             m_v, u_v, v_v, idx_v, w_v, buf_v, out_v,
             sem_in, sem_g, sem_out) = refs
        wid = lax.axis_index("s") * _NC + lax.axis_index("c")
        base_w = wid * _PW
        iota = lax.iota(jnp.int32, _L)

        @pl.loop(0, _NCH)
        def chunk(k):
            cb = base_w + k * _C

            cm = pltpu.async_copy(m_hbm.at[pl.ds(cb, _C)], m_v, sem_in)
            if use_h:
                blk0 = pl.multiple_of(cb // 128, _C // 128)
                ca = pltpu.async_copy(h_hbm.at[pl.ds(blk0, _C // 128)], h_v,
                                      sem_in)
                cm.wait(); ca.wait()
            else:
                ca = pltpu.async_copy(u_hbm.at[pl.ds(cb, _C)], u_v, sem_in)
                cc = pltpu.async_copy(v_hbm.at[pl.ds(cb, _C)], v_v, sem_in)
                cm.wait(); ca.wait(); cc.wait()

            @pl.loop(0, _NG)
            def grp(gi):
                off = gi * _L
                mv = m_v[pl.ds(off, _L)]
                if use_h:
                    ci = h_v[gi >> 3, 0, pl.ds((gi & 7) * _L, _L)]
                    cj = h_v[gi >> 3, 1, pl.ds((gi & 7) * _L, _L)]
                    fi = (ci + 1.0) * (0.5 * I)
                    fj = (cj + 1.0) * (0.5 * J)
                else:
                    fi = u_v[pl.ds(off, _L)] * float(I)
                    fj = v_v[pl.ds(off, _L)] * float(J)
                i1, i2, ir = _corners(fi, I)
                j1, j2, jr = _corners(fj, J)
                base = mv * (I * J)
                a1 = base + i1 * J
                a2 = base + i2 * J
                idx_v[pl.ds(0 * _C + off, _L)] = a1 + j1
                idx_v[pl.ds(1 * _C + off, _L)] = a2 + j1
                idx_v[pl.ds(2 * _C + off, _L)] = a1 + j2
                idx_v[pl.ds(3 * _C + off, _L)] = a2 + j2
                omi = 1.0 - ir
                omj = 1.0 - jr
                w_v[pl.ds(0 * _C + off, _L)] = omi * omj
                w_v[pl.ds(1 * _C + off, _L)] = ir * omj
                w_v[pl.ds(2 * _C + off, _L)] = omi * jr
                w_v[pl.ds(3 * _C + off, _L)] = ir * jr

            @pl.loop(0, _NBLK)
            def fire(b):
                o = b * _IBLK
                pltpu.async_copy(f_hbm.at[idx_v.at[pl.ds(o, _IBLK)]],
                                 buf_v.at[pl.ds(o, _IBLK)], sem_g)

            @pl.when(k > 0)
            def _():
                pltpu.make_async_copy(out_v,
                                      out_hbm.at[pl.ds(0, _C // 128)],
                                      sem_out).wait()

            pltpu.make_async_copy(f_hbm.at[pl.ds(0, _RB)], buf_v,
                                  sem_g).wait()

            @pl.loop(0, _NG)
            def blend(gi):
                off = gi * _L
                pts = off + iota
                w11 = w_v[pl.ds(0 * _C + off, _L)]
                w21 = w_v[pl.ds(1 * _C + off, _L)]
                w12 = w_v[pl.ds(2 * _C + off, _L)]
                w22 = w_v[pl.ds(3 * _C + off, _L)]
                r11 = pts
                r21 = pts + 1 * _C
                r12 = pts + 2 * _C
                r22 = pts + 3 * _C
                for l in range(LD):
                    col = jnp.full((_L,), l, jnp.int32)
                    g11 = plsc.load_gather(buf_v, [r11, col])
                    g21 = plsc.load_gather(buf_v, [r21, col])
                    g12 = plsc.load_gather(buf_v, [r12, col])
                    g22 = plsc.load_gather(buf_v, [r22, col])
                    acc = g11 * w11 + g21 * w21 + g12 * w12 + g22 * w22
                    out_v[gi >> 3, l, pl.ds((gi & 7) * _L, _L)] = acc

            o0 = pl.multiple_of(cb // 128, _C // 128)
            pltpu.async_copy(out_v, out_hbm.at[pl.ds(o0, _C // 128)],
                             sem_out)

        pltpu.make_async_copy(out_v, out_hbm.at[pl.ds(0, _C // 128)],
                              sem_out).wait()

    coord_scratch = ([pltpu.VMEM((_C // 128, 2, 128), jnp.float32)]
                     if use_h else
                     [pltpu.VMEM((_C,), jnp.float32),
                      pltpu.VMEM((_C,), jnp.float32)])
    return pl.kernel(
        body,
        out_type=jax.ShapeDtypeStruct((_N // 128, LD, 128), jnp.float32),
        mesh=plsc.VectorSubcoreMesh(core_axis_name="c",
                                    subcore_axis_name="s"),
        compiler_params=pltpu.CompilerParams(needs_layout_passes=False,
                                             use_tc_tiling_on_sc=False),
        scratch_types=[pltpu.VMEM((_C,), jnp.int32)] + coord_scratch + [
            pltpu.VMEM((_RB,), jnp.int32),
            pltpu.VMEM((_RB,), jnp.float32),
            pltpu.VMEM((_RB, LD), jnp.float32),
            pltpu.VMEM((_C // 128, LD, 128), jnp.float32),
            pltpu.SemaphoreType.DMA,
            pltpu.SemaphoreType.DMA,
            pltpu.SemaphoreType.DMA,
        ],
    )


_xy_kernel = _plane_kernel(_HX, _HY, _LXY, use_h=True)
_uv_kernel = _plane_kernel(_U, _V, _LUV, use_h=False)


@jax.jit
def kernel(m, h, u, v, Fxy, Fuv):
    fxy = Fxy.reshape(_M * _HX * _HY, _LXY)
    fuv = Fuv.reshape(_M * _U * _V, _LUV)
    h3 = h.reshape(_N // 128, 128, 2).transpose(0, 2, 1)
    out_uv = _uv_kernel(m, u, v, fuv)
    out_xy = _xy_kernel(m, h3, fxy)
    out = jnp.concatenate([out_xy, out_uv], axis=1)
    return out.transpose(0, 2, 1).reshape(_N, _LXY + _LUV)


# speedup vs baseline: 2.5909x; 1.1351x over previous
import jax
import jax.numpy as jnp
from jax import lax
from jax.experimental import pallas as pl
from jax.experimental.pallas import tpu as pltpu
from jax.experimental.pallas import tpu_sc as plsc

_M, _HX, _HY, _LXY = 8, 512, 512, 8
_U, _V, _LUV = 400, 400, 8
_N = 1048576

_NC, _NS, _L = 2, 16, 16
_NW = _NC * _NS
_PW = _N // _NW
_C = 512
_NCH = _PW // _C
_NG = _C // _L
_RB = 4 * _C
_IBLK = 128
_NBLK = _RB // _IBLK


def _corners(find, size):
    find = jnp.where(find >= float(size), jnp.full((_L,), float(size - 1)),
                     find)
    i1 = find.astype(jnp.int32)
    fr = find - i1.astype(jnp.float32)
    i2 = i1 + 1
    i2 = jnp.where(i2 >= size, jnp.zeros((_L,), jnp.int32), i2)
    return i1, i2, fr


def _plane_kernel(I, J, LD, use_h):
    def body(*refs):
        if use_h:
            (m_hbm, h_hbm, f_hbm, out_hbm,
             m_v, h_v, idx_v, w_v, buf_v, out_v,
             sem_in, sem_g, sem_out) = refs
        else:
            (m_hbm, u_hbm, v_hbm, f_hbm, out_hbm,
             m_v, u_v, v_v, idx_v, w_v, buf_v, out_v,
             sem_in, sem_g, sem_out) = refs
        wid = lax.axis_index("s") * _NC + lax.axis_index("c")
        base_w = wid * _PW
        iota = lax.iota(jnp.int32, _L)

        @pl.loop(0, _NCH)
        def chunk(k):
            cb = base_w + k * _C

            cm = pltpu.async_copy(m_hbm.at[pl.ds(cb, _C)], m_v, sem_in)
            if use_h:
                blk0 = pl.multiple_of(cb // 128, _C // 128)
                ca = pltpu.async_copy(h_hbm.at[pl.ds(blk0, _C // 128)], h_v,
                                      sem_in)
                cm.wait(); ca.wait()
            else:
                ca = pltpu.async_copy(u_hbm.at[pl.ds(cb, _C)], u_v, sem_in)
                cc = pltpu.async_copy(v_hbm.at[pl.ds(cb, _C)], v_v, sem_in)
                cm.wait(); ca.wait(); cc.wait()

            @pl.loop(0, _NG)
            def grp(gi):
                off = gi * _L
                mv = m_v[pl.ds(off, _L)]
                if use_h:
                    ci = h_v[gi >> 3, 0, pl.ds((gi & 7) * _L, _L)]
                    cj = h_v[gi >> 3, 1, pl.ds((gi & 7) * _L, _L)]
                    fi = (ci + 1.0) * (0.5 * I)
                    fj = (cj + 1.0) * (0.5 * J)
                else:
                    fi = u_v[pl.ds(off, _L)] * float(I)
                    fj = v_v[pl.ds(off, _L)] * float(J)
                i1, i2, ir = _corners(fi, I)
                j1, j2, jr = _corners(fj, J)
                base = mv * (I * J)
                a1 = base + i1 * J
                a2 = base + i2 * J
                idx_v[pl.ds(0 * _C + off, _L)] = a1 + j1
                idx_v[pl.ds(1 * _C + off, _L)] = a2 + j1
                idx_v[pl.ds(2 * _C + off, _L)] = a1 + j2
                idx_v[pl.ds(3 * _C + off, _L)] = a2 + j2
                omi = 1.0 - ir
                omj = 1.0 - jr
                w_v[pl.ds(0 * _C + off, _L)] = omi * omj
                w_v[pl.ds(1 * _C + off, _L)] = ir * omj
                w_v[pl.ds(2 * _C + off, _L)] = omi * jr
                w_v[pl.ds(3 * _C + off, _L)] = ir * jr

            @pl.loop(0, _NBLK)
            def fire(b):
                o = b * _IBLK
                pltpu.async_copy(f_hbm.at[idx_v.at[pl.ds(o, _IBLK)]],
                                 buf_v.at[pl.ds(o, _IBLK)], sem_g)

            @pl.when(k > 0)
            def _():
                pltpu.make_async_copy(out_v,
                                      out_hbm.at[pl.ds(0, _C // 128)],
                                      sem_out).wait()

            pltpu.make_async_copy(f_hbm.at[pl.ds(0, _RB)], buf_v,
                                  sem_g).wait()

            @pl.loop(0, _NG)
            def blend(gi):
                off = gi * _L
                pts = off + iota
                w11 = w_v[pl.ds(0 * _C + off, _L)]
                w21 = w_v[pl.ds(1 * _C + off, _L)]
                w12 = w_v[pl.ds(2 * _C + off, _L)]
                w22 = w_v[pl.ds(3 * _C + off, _L)]
                r11 = pts
                r21 = pts + 1 * _C
                r12 = pts + 2 * _C
                r22 = pts + 3 * _C
                for l in range(LD):
                    col = jnp.full((_L,), l, jnp.int32)
                    g11 = plsc.load_gather(buf_v, [r11, col])
                    g21 = plsc.load_gather(buf_v, [r21, col])
                    g12 = plsc.load_gather(buf_v, [r12, col])
                    g22 = plsc.load_gather(buf_v, [r22, col])
                    acc = g11 * w11 + g21 * w21 + g12 * w12 + g22 * w22
                    out_v[gi >> 3, l, pl.ds((gi & 7) * _L, _L)] = acc

            o0 = pl.multiple_of(cb // 128, _C // 128)
            pltpu.async_copy(out_v, out_hbm.at[pl.ds(o0, _C // 128)],
                             sem_out)

        pltpu.make_async_copy(out_v, out_hbm.at[pl.ds(0, _C // 128)],
                              sem_out).wait()

    coord_scratch = ([pltpu.VMEM((_C // 128, 2, 128), jnp.float32)]
                     if use_h else
                     [pltpu.VMEM((_C,), jnp.float32),
                      pltpu.VMEM((_C,), jnp.float32)])
    return pl.kernel(
        body,
        out_type=jax.ShapeDtypeStruct((_N // 128, LD, 128), jnp.float32),
        mesh=plsc.VectorSubcoreMesh(core_axis_name="c",
                                    subcore_axis_name="s"),
        compiler_params=pltpu.CompilerParams(needs_layout_passes=False,
                                             use_tc_tiling_on_sc=False),
        scratch_types=[pltpu.VMEM((_C,), jnp.int32)] + coord_scratch + [
            pltpu.VMEM((_RB,), jnp.int32),
            pltpu.VMEM((_RB,), jnp.float32),
            pltpu.VMEM((_RB, LD), jnp.float32),
            pltpu.VMEM((_C // 128, LD, 128), jnp.float32),
            pltpu.SemaphoreType.DMA,
            pltpu.SemaphoreType.DMA,
            pltpu.SemaphoreType.DMA,
        ],
    )


_xy_kernel = _plane_kernel(_HX, _HY, _LXY, use_h=True)
_uv_kernel = _plane_kernel(_U, _V, _LUV, use_h=False)

_TT = _M * _HX * (_HY // 128)
_TPW = _TT // _NW
_TB = 8
_TSTEPS = _TPW // _TB


def _tr_body(tin_hbm, tout_hbm, tin0, tin1, tout_v, sem_i, sem_o):
    wid = lax.axis_index("s") * _NC + lax.axis_index("c")
    tbase = wid * _TPW
    iota = lax.iota(jnp.int32, _L)
    d1 = iota & 7
    d2base = iota >> 3

    pltpu.async_copy(tin_hbm.at[pl.ds(tbase, _TB)], tin0, sem_i)

    @pl.loop(0, _TSTEPS)
    def step(c):
        tb = tbase + c * _TB

        @pl.when(c + 1 < _TSTEPS)
        def _():
            @pl.when(lax.rem(c, 2) == 0)
            def _():
                pltpu.async_copy(tin_hbm.at[pl.ds(tb + _TB, _TB)], tin1,
                                 sem_i)

            @pl.when(lax.rem(c, 2) == 1)
            def _():
                pltpu.async_copy(tin_hbm.at[pl.ds(tb + _TB, _TB)], tin0,
                                 sem_i)

        pltpu.make_async_copy(tin_hbm.at[pl.ds(0, _TB)], tin0, sem_i).wait()

        @pl.when(c > 0)
        def _():
            pltpu.make_async_copy(tout_v, tout_hbm.at[pl.ds(0, _TB)],
                                  sem_o).wait()

        for par in range(2):
            tin = (tin0, tin1)[par]

            @pl.when(lax.rem(c, 2) == par)
            def _():
                for t in range(_TB):
                    d0 = jnp.full((_L,), t, jnp.int32)
                    for g in range(64):
                        vals = plsc.load_gather(tin, [d0, d1, d2base + 2 * g])
                        tout_v[t, g, :] = vals

        pltpu.async_copy(tout_v, tout_hbm.at[pl.ds(tb, _TB)], sem_o)

    pltpu.make_async_copy(tout_v, tout_hbm.at[pl.ds(0, _TB)], sem_o).wait()


_xy_transpose = pl.kernel(
    _tr_body,
    out_type=jax.ShapeDtypeStruct((_TT, 64, 16), jnp.float32),
    mesh=plsc.VectorSubcoreMesh(core_axis_name="c", subcore_axis_name="s"),
    compiler_params=pltpu.CompilerParams(needs_layout_passes=False,
                                         use_tc_tiling_on_sc=False),
    scratch_types=[
        pltpu.VMEM((_TB, 8, 128), jnp.float32),
        pltpu.VMEM((_TB, 8, 128), jnp.float32),
        pltpu.VMEM((_TB, 64, 16), jnp.float32),
        pltpu.SemaphoreType.DMA,
        pltpu.SemaphoreType.DMA,
    ],
)


@jax.jit
def kernel(m, h, u, v, Fxy, Fuv):
    fxy3 = (Fxy.transpose(0, 1, 3, 2)
            .reshape(_M, _HX, _LXY, _HY // 128, 128)
            .transpose(0, 1, 3, 2, 4)
            .reshape(_TT, 8, 128))
    fxy = _xy_transpose(fxy3).reshape(_M * _HX * _HY, _LXY)
    fuv = Fuv.reshape(_M * _U * _V, _LUV)
    h3 = h.reshape(_N // 128, 128, 2).transpose(0, 2, 1)
    out_xy = _xy_kernel(m, h3, fxy)
    out_uv = _uv_kernel(m, u, v, fuv)
    out = jnp.concatenate([out_xy, out_uv], axis=1)
    return out.transpose(0, 2, 1).reshape(_N, _LXY + _LUV)


# speedup vs baseline: 2.9367x; 1.1335x over previous
import jax
import jax.numpy as jnp
from jax import lax
from jax.experimental import pallas as pl
from jax.experimental.pallas import tpu as pltpu
from jax.experimental.pallas import tpu_sc as plsc

_M, _HX, _HY, _LXY = 8, 512, 512, 8
_U, _V, _LUV = 400, 400, 8
_N = 1048576

_NC, _NS, _L = 2, 16, 16
_NW = _NC * _NS
_PW = _N // _NW
_C = 512
_NCH = _PW // _C
_NG = _C // _L
_RB = 4 * _C
_IBLK = 128
_NBLK = _RB // _IBLK


def _corners(find, size):
    find = jnp.where(find >= float(size), jnp.full((_L,), float(size - 1)),
                     find)
    i1 = find.astype(jnp.int32)
    fr = find - i1.astype(jnp.float32)
    i2 = i1 + 1
    i2 = jnp.where(i2 >= size, jnp.zeros((_L,), jnp.int32), i2)
    return i1, i2, fr


def _plane_kernel(I, J, LD, use_h):
    def body(*refs):
        if use_h:
            (m_hbm, h_hbm, f_hbm, out_hbm,
             m_v, h_v, idx_v, w_v, buf_v, out_v,
             sem_in, sem_g, sem_out) = refs
        else:
            (m_hbm, u_hbm, v_hbm, f_hbm, out_hbm,
             m_v, u_v, v_v, idx_v, w_v, buf_v, out_v,
             sem_in, sem_g, sem_out) = refs
        wid = lax.axis_index("s") * _NC + lax.axis_index("c")
        base_w = wid * _PW
        iota = lax.iota(jnp.int32, _L)

        @pl.loop(0, _NCH)
        def chunk(k):
            cb = base_w + k * _C

            cm = pltpu.async_copy(m_hbm.at[pl.ds(cb, _C)], m_v, sem_in)
            if use_h:
                blk0 = pl.multiple_of(cb // 128, _C // 128)
                ca = pltpu.async_copy(h_hbm.at[pl.ds(blk0, _C // 128)], h_v,
                                      sem_in)
                cm.wait(); ca.wait()
            else:
                ca = pltpu.async_copy(u_hbm.at[pl.ds(cb, _C)], u_v, sem_in)
                cc = pltpu.async_copy(v_hbm.at[pl.ds(cb, _C)], v_v, sem_in)
                cm.wait(); ca.wait(); cc.wait()

            @pl.loop(0, _NG)
            def grp(gi):
                off = gi * _L
                mv = m_v[pl.ds(off, _L)]
                if use_h:
                    ci = h_v[gi >> 3, 0, pl.ds((gi & 7) * _L, _L)]
                    cj = h_v[gi >> 3, 1, pl.ds((gi & 7) * _L, _L)]
                    fi = (ci + 1.0) * (0.5 * I)
                    fj = (cj + 1.0) * (0.5 * J)
                else:
                    fi = u_v[pl.ds(off, _L)] * float(I)
                    fj = v_v[pl.ds(off, _L)] * float(J)
                i1, i2, ir = _corners(fi, I)
                j1, j2, jr = _corners(fj, J)
                base = mv * (I * J)
                a1 = base + i1 * J
                a2 = base + i2 * J
                idx_v[pl.ds(0 * _C + off, _L)] = a1 + j1
                idx_v[pl.ds(1 * _C + off, _L)] = a2 + j1
                idx_v[pl.ds(2 * _C + off, _L)] = a1 + j2
                idx_v[pl.ds(3 * _C + off, _L)] = a2 + j2
                omi = 1.0 - ir
                omj = 1.0 - jr
                w_v[pl.ds(0 * _C + off, _L)] = omi * omj
                w_v[pl.ds(1 * _C + off, _L)] = ir * omj
                w_v[pl.ds(2 * _C + off, _L)] = omi * jr
                w_v[pl.ds(3 * _C + off, _L)] = ir * jr

            @pl.loop(0, _NBLK)
            def fire(b):
                o = b * _IBLK
                pltpu.async_copy(f_hbm.at[idx_v.at[pl.ds(o, _IBLK)]],
                                 buf_v.at[pl.ds(o, _IBLK)], sem_g)

            @pl.when(k > 0)
            def _():
                pltpu.make_async_copy(out_v,
                                      out_hbm.at[pl.ds(0, _C // 128)],
                                      sem_out).wait()

            pltpu.make_async_copy(f_hbm.at[pl.ds(0, _RB)], buf_v,
                                  sem_g).wait()

            @pl.loop(0, _NG)
            def blend(gi):
                off = gi * _L
                pts = off + iota
                w11 = w_v[pl.ds(0 * _C + off, _L)]
                w21 = w_v[pl.ds(1 * _C + off, _L)]
                w12 = w_v[pl.ds(2 * _C + off, _L)]
                w22 = w_v[pl.ds(3 * _C + off, _L)]
                r11 = pts
                r21 = pts + 1 * _C
                r12 = pts + 2 * _C
                r22 = pts + 3 * _C
                for l in range(LD):
                    col = jnp.full((_L,), l, jnp.int32)
                    g11 = plsc.load_gather(buf_v, [r11, col])
                    g21 = plsc.load_gather(buf_v, [r21, col])
                    g12 = plsc.load_gather(buf_v, [r12, col])
                    g22 = plsc.load_gather(buf_v, [r22, col])
                    acc = g11 * w11 + g21 * w21 + g12 * w12 + g22 * w22
                    out_v[gi >> 3, l, pl.ds((gi & 7) * _L, _L)] = acc

            o0 = pl.multiple_of(cb // 128, _C // 128)
            pltpu.async_copy(out_v, out_hbm.at[pl.ds(o0, _C // 128)],
                             sem_out)

        pltpu.make_async_copy(out_v, out_hbm.at[pl.ds(0, _C // 128)],
                              sem_out).wait()

    coord_scratch = ([pltpu.VMEM((_C // 128, 2, 128), jnp.float32)]
                     if use_h else
                     [pltpu.VMEM((_C,), jnp.float32),
                      pltpu.VMEM((_C,), jnp.float32)])
    return pl.kernel(
        body,
        out_type=jax.ShapeDtypeStruct((_N // 128, LD, 128), jnp.float32),
        mesh=plsc.VectorSubcoreMesh(core_axis_name="c",
                                    subcore_axis_name="s"),
        compiler_params=pltpu.CompilerParams(needs_layout_passes=False,
                                             use_tc_tiling_on_sc=False),
        scratch_types=[pltpu.VMEM((_C,), jnp.int32)] + coord_scratch + [
            pltpu.VMEM((_RB,), jnp.int32),
            pltpu.VMEM((_RB,), jnp.float32),
            pltpu.VMEM((_RB, LD), jnp.float32),
            pltpu.VMEM((_C // 128, LD, 128), jnp.float32),
            pltpu.SemaphoreType.DMA,
            pltpu.SemaphoreType.DMA,
            pltpu.SemaphoreType.DMA,
        ],
    )


_xy_kernel = _plane_kernel(_HX, _HY, _LXY, use_h=True)
_uv_kernel = _plane_kernel(_U, _V, _LUV, use_h=False)


def _both_body(m_hbm, h_hbm, u_hbm, v_hbm, fxy_hbm, fuv_hbm, out_hbm,
               m_v, h_v, u_v, v_v, idxxy_v, idxuv_v, wxy_v, wuv_v,
               bufxy_v, bufuv_v, out_v, sem_in, sem_xy, sem_uv, sem_out):
    wid = lax.axis_index("s") * _NC + lax.axis_index("c")
    base_w = wid * _PW
    iota = lax.iota(jnp.int32, _L)

    @pl.loop(0, _NCH)
    def chunk(k):
        cb = base_w + k * _C
        blk0 = pl.multiple_of(cb // 128, _C // 128)

        cm = pltpu.async_copy(m_hbm.at[pl.ds(cb, _C)], m_v, sem_in)
        ch = pltpu.async_copy(h_hbm.at[pl.ds(blk0, _C // 128)], h_v, sem_in)
        cu = pltpu.async_copy(u_hbm.at[pl.ds(cb, _C)], u_v, sem_in)
        cv = pltpu.async_copy(v_hbm.at[pl.ds(cb, _C)], v_v, sem_in)
        cm.wait(); ch.wait(); cu.wait(); cv.wait()

        @pl.loop(0, _NG)
        def grp(gi):
            off = gi * _L
            mv = m_v[pl.ds(off, _L)]
            ci = h_v[gi >> 3, 0, pl.ds((gi & 7) * _L, _L)]
            cj = h_v[gi >> 3, 1, pl.ds((gi & 7) * _L, _L)]
            i1, i2, ir = _corners((ci + 1.0) * (0.5 * _HX), _HX)
            j1, j2, jr = _corners((cj + 1.0) * (0.5 * _HY), _HY)
            base = mv * (_HX * _HY)
            a1 = base + i1 * _HY
            a2 = base + i2 * _HY
            idxxy_v[pl.ds(0 * _C + off, _L)] = a1 + j1
            idxxy_v[pl.ds(1 * _C + off, _L)] = a2 + j1
            idxxy_v[pl.ds(2 * _C + off, _L)] = a1 + j2
            idxxy_v[pl.ds(3 * _C + off, _L)] = a2 + j2
            omi = 1.0 - ir
            omj = 1.0 - jr
            wxy_v[pl.ds(0 * _C + off, _L)] = omi * omj
            wxy_v[pl.ds(1 * _C + off, _L)] = ir * omj
            wxy_v[pl.ds(2 * _C + off, _L)] = omi * jr
            wxy_v[pl.ds(3 * _C + off, _L)] = ir * jr

            p1, p2, pr = _corners(u_v[pl.ds(off, _L)] * float(_U), _U)
            q1, q2, qr = _corners(v_v[pl.ds(off, _L)] * float(_V), _V)
            baseu = mv * (_U * _V)
            b1 = baseu + p1 * _V
            b2 = baseu + p2 * _V
            idxuv_v[pl.ds(0 * _C + off, _L)] = b1 + q1
            idxuv_v[pl.ds(1 * _C + off, _L)] = b2 + q1
            idxuv_v[pl.ds(2 * _C + off, _L)] = b1 + q2
            idxuv_v[pl.ds(3 * _C + off, _L)] = b2 + q2
            omp = 1.0 - pr
            omq = 1.0 - qr
            wuv_v[pl.ds(0 * _C + off, _L)] = omp * omq
            wuv_v[pl.ds(1 * _C + off, _L)] = pr * omq
            wuv_v[pl.ds(2 * _C + off, _L)] = omp * qr
            wuv_v[pl.ds(3 * _C + off, _L)] = pr * qr

        @pl.loop(0, _NBLK)
        def fire(b):
            o = b * _IBLK
            pltpu.async_copy(fxy_hbm.at[idxxy_v.at[pl.ds(o, _IBLK)]],
                             bufxy_v.at[pl.ds(o, _IBLK)], sem_xy)
            pltpu.async_copy(fuv_hbm.at[idxuv_v.at[pl.ds(o, _IBLK)]],
                             bufuv_v.at[pl.ds(o, _IBLK)], sem_uv)

        @pl.when(k > 0)
        def _():
            pltpu.make_async_copy(out_v, out_hbm.at[pl.ds(0, _C // 128)],
                                  sem_out).wait()

        pltpu.make_async_copy(fxy_hbm.at[pl.ds(0, _RB)], bufxy_v,
                              sem_xy).wait()
        pltpu.make_async_copy(fuv_hbm.at[pl.ds(0, _RB)], bufuv_v,
                              sem_uv).wait()

        @pl.loop(0, _NG)
        def blend(gi):
            off = gi * _L
            pts = off + iota
            for (buf, wv, cbase) in ((bufxy_v, wxy_v, 0),
                                     (bufuv_v, wuv_v, _LXY)):
                w11 = wv[pl.ds(0 * _C + off, _L)]
                w21 = wv[pl.ds(1 * _C + off, _L)]
                w12 = wv[pl.ds(2 * _C + off, _L)]
                w22 = wv[pl.ds(3 * _C + off, _L)]
                r11 = pts
                r21 = pts + 1 * _C
                r12 = pts + 2 * _C
                r22 = pts + 3 * _C
                for l in range(_LXY):
                    col = jnp.full((_L,), l, jnp.int32)
                    g11 = plsc.load_gather(buf, [r11, col])
                    g21 = plsc.load_gather(buf, [r21, col])
                    g12 = plsc.load_gather(buf, [r12, col])
                    g22 = plsc.load_gather(buf, [r22, col])
                    acc = g11 * w11 + g21 * w21 + g12 * w12 + g22 * w22
                    out_v[gi >> 3, cbase + l, pl.ds((gi & 7) * _L, _L)] = acc

        o0 = pl.multiple_of(cb // 128, _C // 128)
        pltpu.async_copy(out_v, out_hbm.at[pl.ds(o0, _C // 128)], sem_out)

    pltpu.make_async_copy(out_v, out_hbm.at[pl.ds(0, _C // 128)],
                          sem_out).wait()


_both_kernel = pl.kernel(
    _both_body,
    out_type=jax.ShapeDtypeStruct((_N // 128, 16, 128), jnp.float32),
    mesh=plsc.VectorSubcoreMesh(core_axis_name="c", subcore_axis_name="s"),
    compiler_params=pltpu.CompilerParams(needs_layout_passes=False,
                                         use_tc_tiling_on_sc=False),
    scratch_types=[
        pltpu.VMEM((_C,), jnp.int32),
        pltpu.VMEM((_C // 128, 2, 128), jnp.float32),
        pltpu.VMEM((_C,), jnp.float32),
        pltpu.VMEM((_C,), jnp.float32),
        pltpu.VMEM((_RB,), jnp.int32),
        pltpu.VMEM((_RB,), jnp.int32),
        pltpu.VMEM((_RB,), jnp.float32),
        pltpu.VMEM((_RB,), jnp.float32),
        pltpu.VMEM((_RB, _LXY), jnp.float32),
        pltpu.VMEM((_RB, _LUV), jnp.float32),
        pltpu.VMEM((_C // 128, 16, 128), jnp.float32),
        pltpu.SemaphoreType.DMA,
        pltpu.SemaphoreType.DMA,
        pltpu.SemaphoreType.DMA,
        pltpu.SemaphoreType.DMA,
    ],
)


_TT = _M * _HX * (_HY // 128)
_TPW = _TT // _NW
_TB = 8
_TSTEPS = _TPW // _TB


def _tr_body(tin_hbm, tout_hbm, tin0, tin1, tout_v, sem_i, sem_o):
    wid = lax.axis_index("s") * _NC + lax.axis_index("c")
    tbase = wid * _TPW
    iota = lax.iota(jnp.int32, _L)
    d1 = iota & 7
    d2base = iota >> 3

    pltpu.async_copy(tin_hbm.at[pl.ds(tbase, _TB)], tin0, sem_i)

    @pl.loop(0, _TSTEPS)
    def step(c):
        tb = tbase + c * _TB

        @pl.when(c + 1 < _TSTEPS)
        def _():
            @pl.when(lax.rem(c, 2) == 0)
            def _():
                pltpu.async_copy(tin_hbm.at[pl.ds(tb + _TB, _TB)], tin1,
                                 sem_i)

            @pl.when(lax.rem(c, 2) == 1)
            def _():
                pltpu.async_copy(tin_hbm.at[pl.ds(tb + _TB, _TB)], tin0,
                                 sem_i)

        pltpu.make_async_copy(tin_hbm.at[pl.ds(0, _TB)], tin0, sem_i).wait()

        @pl.when(c > 0)
        def _():
            pltpu.make_async_copy(tout_v, tout_hbm.at[pl.ds(0, _TB)],
                                  sem_o).wait()

        for par in range(2):
            tin = (tin0, tin1)[par]

            @pl.when(lax.rem(c, 2) == par)
            def _():
                for t in range(_TB):
                    d0 = jnp.full((_L,), t, jnp.int32)
                    for g in range(64):
                        vals = plsc.load_gather(tin, [d0, d1, d2base + 2 * g])
                        tout_v[t, g, :] = vals

        pltpu.async_copy(tout_v, tout_hbm.at[pl.ds(tb, _TB)], sem_o)

    pltpu.make_async_copy(tout_v, tout_hbm.at[pl.ds(0, _TB)], sem_o).wait()


_xy_transpose = pl.kernel(
    _tr_body,
    out_type=jax.ShapeDtypeStruct((_TT, 64, 16), jnp.float32),
    mesh=plsc.VectorSubcoreMesh(core_axis_name="c", subcore_axis_name="s"),
    compiler_params=pltpu.CompilerParams(needs_layout_passes=False,
                                         use_tc_tiling_on_sc=False),
    scratch_types=[
        pltpu.VMEM((_TB, 8, 128), jnp.float32),
        pltpu.VMEM((_TB, 8, 128), jnp.float32),
        pltpu.VMEM((_TB, 64, 16), jnp.float32),
        pltpu.SemaphoreType.DMA,
        pltpu.SemaphoreType.DMA,
    ],
)


@jax.jit
def kernel(m, h, u, v, Fxy, Fuv):
    fxy3 = (Fxy.transpose(0, 1, 3, 2)
            .reshape(_M, _HX, _LXY, _HY // 128, 128)
            .transpose(0, 1, 3, 2, 4)
            .reshape(_TT, 8, 128))
    fxy = _xy_transpose(fxy3).reshape(_M * _HX * _HY, _LXY)
    fuv = Fuv.reshape(_M * _U * _V, _LUV)
    h3 = h.reshape(_N // 128, 128, 2).transpose(0, 2, 1)
    out = _both_kernel(m, h3, u, v, fxy, fuv)
    return out.transpose(0, 2, 1).reshape(_N, _LXY + _LUV)


# speedup vs baseline: 3.5888x; 1.2220x over previous
import jax
import jax.numpy as jnp
from jax import lax
from jax.experimental import pallas as pl
from jax.experimental.pallas import tpu as pltpu
from jax.experimental.pallas import tpu_sc as plsc

_M, _HX, _HY, _LXY = 8, 512, 512, 8
_U, _V, _LUV = 400, 400, 8
_N = 1048576

_NC, _NS, _L = 2, 16, 16
_NW = _NC * _NS
_PW = _N // _NW
_C = 512
_NCH = _PW // _C
_NG = _C // _L
_RB = 4 * _C
_IBLK = 128
_NBLK = _RB // _IBLK


def _corners(find, size):
    find = jnp.where(find >= float(size), jnp.full((_L,), float(size - 1)),
                     find)
    i1 = find.astype(jnp.int32)
    fr = find - i1.astype(jnp.float32)
    i2 = i1 + 1
    i2 = jnp.where(i2 >= size, jnp.zeros((_L,), jnp.int32), i2)
    return i1, i2, fr


def _plane_kernel(I, J, LD, use_h):
    def body(*refs):
        if use_h:
            (m_hbm, h_hbm, f_hbm, out_hbm,
             m_v, h_v, idx_v, w_v, buf_v, out_v,
             sem_in, sem_g, sem_out) = refs
        else:
            (m_hbm, u_hbm, v_hbm, f_hbm, out_hbm,
             m_v, u_v, v_v, idx_v, w_v, buf_v, out_v,
             sem_in, sem_g, sem_out) = refs
        wid = lax.axis_index("s") * _NC + lax.axis_index("c")
        base_w = wid * _PW
        iota = lax.iota(jnp.int32, _L)

        @pl.loop(0, _NCH)
        def chunk(k):
            cb = base_w + k * _C

            cm = pltpu.async_copy(m_hbm.at[pl.ds(cb, _C)], m_v, sem_in)
            if use_h:
                blk0 = pl.multiple_of(cb // 128, _C // 128)
                ca = pltpu.async_copy(h_hbm.at[pl.ds(blk0, _C // 128)], h_v,
                                      sem_in)
                cm.wait(); ca.wait()
            else:
                ca = pltpu.async_copy(u_hbm.at[pl.ds(cb, _C)], u_v, sem_in)
                cc = pltpu.async_copy(v_hbm.at[pl.ds(cb, _C)], v_v, sem_in)
                cm.wait(); ca.wait(); cc.wait()

            @pl.loop(0, _NG)
            def grp(gi):
                off = gi * _L
                mv = m_v[pl.ds(off, _L)]
                if use_h:
                    ci = h_v[gi >> 3, 0, pl.ds((gi & 7) * _L, _L)]
                    cj = h_v[gi >> 3, 1, pl.ds((gi & 7) * _L, _L)]
                    fi = (ci + 1.0) * (0.5 * I)
                    fj = (cj + 1.0) * (0.5 * J)
                else:
                    fi = u_v[pl.ds(off, _L)] * float(I)
                    fj = v_v[pl.ds(off, _L)] * float(J)
                i1, i2, ir = _corners(fi, I)
                j1, j2, jr = _corners(fj, J)
                base = mv * (I * J)
                a1 = base + i1 * J
                a2 = base + i2 * J
                idx_v[pl.ds(0 * _C + off, _L)] = a1 + j1
                idx_v[pl.ds(1 * _C + off, _L)] = a2 + j1
                idx_v[pl.ds(2 * _C + off, _L)] = a1 + j2
                idx_v[pl.ds(3 * _C + off, _L)] = a2 + j2
                omi = 1.0 - ir
                omj = 1.0 - jr
                w_v[pl.ds(0 * _C + off, _L)] = omi * omj
                w_v[pl.ds(1 * _C + off, _L)] = ir * omj
                w_v[pl.ds(2 * _C + off, _L)] = omi * jr
                w_v[pl.ds(3 * _C + off, _L)] = ir * jr

            @pl.loop(0, _NBLK)
            def fire(b):
                o = b * _IBLK
                pltpu.async_copy(f_hbm.at[idx_v.at[pl.ds(o, _IBLK)]],
                                 buf_v.at[pl.ds(o, _IBLK)], sem_g)

            @pl.when(k > 0)
            def _():
                pltpu.make_async_copy(out_v,
                                      out_hbm.at[pl.ds(0, _C // 128)],
                                      sem_out).wait()

            pltpu.make_async_copy(f_hbm.at[pl.ds(0, _RB)], buf_v,
                                  sem_g).wait()

            @pl.loop(0, _NG)
            def blend(gi):
                off = gi * _L
                pts = off + iota
                w11 = w_v[pl.ds(0 * _C + off, _L)]
                w21 = w_v[pl.ds(1 * _C + off, _L)]
                w12 = w_v[pl.ds(2 * _C + off, _L)]
                w22 = w_v[pl.ds(3 * _C + off, _L)]
                r11 = pts
                r21 = pts + 1 * _C
                r12 = pts + 2 * _C
                r22 = pts + 3 * _C
                for l in range(LD):
                    col = jnp.full((_L,), l, jnp.int32)
                    g11 = plsc.load_gather(buf_v, [r11, col])
                    g21 = plsc.load_gather(buf_v, [r21, col])
                    g12 = plsc.load_gather(buf_v, [r12, col])
                    g22 = plsc.load_gather(buf_v, [r22, col])
                    acc = g11 * w11 + g21 * w21 + g12 * w12 + g22 * w22
                    out_v[gi >> 3, l, pl.ds((gi & 7) * _L, _L)] = acc

            o0 = pl.multiple_of(cb // 128, _C // 128)
            pltpu.async_copy(out_v, out_hbm.at[pl.ds(o0, _C // 128)],
                             sem_out)

        pltpu.make_async_copy(out_v, out_hbm.at[pl.ds(0, _C // 128)],
                              sem_out).wait()

    coord_scratch = ([pltpu.VMEM((_C // 128, 2, 128), jnp.float32)]
                     if use_h else
                     [pltpu.VMEM((_C,), jnp.float32),
                      pltpu.VMEM((_C,), jnp.float32)])
    return pl.kernel(
        body,
        out_type=jax.ShapeDtypeStruct((_N // 128, LD, 128), jnp.float32),
        mesh=plsc.VectorSubcoreMesh(core_axis_name="c",
                                    subcore_axis_name="s"),
        compiler_params=pltpu.CompilerParams(needs_layout_passes=False,
                                             use_tc_tiling_on_sc=False),
        scratch_types=[pltpu.VMEM((_C,), jnp.int32)] + coord_scratch + [
            pltpu.VMEM((_RB,), jnp.int32),
            pltpu.VMEM((_RB,), jnp.float32),
            pltpu.VMEM((_RB, LD), jnp.float32),
            pltpu.VMEM((_C // 128, LD, 128), jnp.float32),
            pltpu.SemaphoreType.DMA,
            pltpu.SemaphoreType.DMA,
            pltpu.SemaphoreType.DMA,
        ],
    )


_xy_kernel = _plane_kernel(_HX, _HY, _LXY, use_h=True)
_uv_kernel = _plane_kernel(_U, _V, _LUV, use_h=False)


def _both_body(m_hbm, h_hbm, u_hbm, v_hbm, fxy_hbm, fuv_hbm, out_hbm,
               m_v, h_v, u_v, v_v, idxxy_v, idxuv_v, wxy_v, wuv_v,
               bufxy_v, bufuv_v, out_v, sem_in, sem_xy, sem_uv, sem_out):
    wid = lax.axis_index("s") * _NC + lax.axis_index("c")
    base_w = wid * _PW
    iota = lax.iota(jnp.int32, _L)
    _OT = _C // 128

    def fire_coords(k):
        cb = base_w + k * _C
        par = lax.rem(k, 2)
        blk0 = pl.multiple_of(cb // 128, _OT)
        pltpu.async_copy(m_hbm.at[pl.ds(cb, _C)],
                         m_v.at[pl.ds(par * _C, _C)], sem_in)
        pltpu.async_copy(h_hbm.at[pl.ds(blk0, _OT)],
                         h_v.at[pl.ds(par * _OT, _OT)], sem_in)
        pltpu.async_copy(u_hbm.at[pl.ds(cb, _C)],
                         u_v.at[pl.ds(par * _C, _C)], sem_in)
        pltpu.async_copy(v_hbm.at[pl.ds(cb, _C)],
                         v_v.at[pl.ds(par * _C, _C)], sem_in)

    def wait_coords():
        pltpu.make_async_copy(m_hbm.at[pl.ds(0, _C)],
                              m_v.at[pl.ds(0, _C)], sem_in).wait()
        pltpu.make_async_copy(h_hbm.at[pl.ds(0, _OT)],
                              h_v.at[pl.ds(0, _OT)], sem_in).wait()
        pltpu.make_async_copy(u_hbm.at[pl.ds(0, _C)],
                              u_v.at[pl.ds(0, _C)], sem_in).wait()
        pltpu.make_async_copy(v_hbm.at[pl.ds(0, _C)],
                              v_v.at[pl.ds(0, _C)], sem_in).wait()

    def pass1_and_fire(k):
        par = lax.rem(k, 2)
        pc = par * _C
        pr = par * _RB

        @pl.loop(0, _NG)
        def grp(gi):
            off = gi * _L
            mv = m_v[pl.ds(pc + off, _L)]
            hrow = par * (_C // 128) + (gi >> 3)
            ci = h_v[hrow, 0, pl.ds((gi & 7) * _L, _L)]
            cj = h_v[hrow, 1, pl.ds((gi & 7) * _L, _L)]
            i1, i2, ir = _corners((ci + 1.0) * (0.5 * _HX), _HX)
            j1, j2, jr = _corners((cj + 1.0) * (0.5 * _HY), _HY)
            base = mv * (_HX * _HY)
            a1 = base + i1 * _HY
            a2 = base + i2 * _HY
            idxxy_v[pl.ds(pr + 0 * _C + off, _L)] = a1 + j1
            idxxy_v[pl.ds(pr + 1 * _C + off, _L)] = a2 + j1
            idxxy_v[pl.ds(pr + 2 * _C + off, _L)] = a1 + j2
            idxxy_v[pl.ds(pr + 3 * _C + off, _L)] = a2 + j2
            omi = 1.0 - ir
            omj = 1.0 - jr
            wxy_v[pl.ds(pr + 0 * _C + off, _L)] = omi * omj
            wxy_v[pl.ds(pr + 1 * _C + off, _L)] = ir * omj
            wxy_v[pl.ds(pr + 2 * _C + off, _L)] = omi * jr
            wxy_v[pl.ds(pr + 3 * _C + off, _L)] = ir * jr

            p1, p2, prf = _corners(u_v[pl.ds(pc + off, _L)] * float(_U), _U)
            q1, q2, qrf = _corners(v_v[pl.ds(pc + off, _L)] * float(_V), _V)
            baseu = mv * (_U * _V)
            b1 = baseu + p1 * _V
            b2 = baseu + p2 * _V
            idxuv_v[pl.ds(pr + 0 * _C + off, _L)] = b1 + q1
            idxuv_v[pl.ds(pr + 1 * _C + off, _L)] = b2 + q1
            idxuv_v[pl.ds(pr + 2 * _C + off, _L)] = b1 + q2
            idxuv_v[pl.ds(pr + 3 * _C + off, _L)] = b2 + q2
            omp = 1.0 - prf
            omq = 1.0 - qrf
            wuv_v[pl.ds(pr + 0 * _C + off, _L)] = omp * omq
            wuv_v[pl.ds(pr + 1 * _C + off, _L)] = prf * omq
            wuv_v[pl.ds(pr + 2 * _C + off, _L)] = omp * qrf
            wuv_v[pl.ds(pr + 3 * _C + off, _L)] = prf * qrf

        @pl.loop(0, _NBLK)
        def fire(b):
            o = pr + b * _IBLK
            pltpu.async_copy(fxy_hbm.at[idxxy_v.at[pl.ds(o, _IBLK)]],
                             bufxy_v.at[pl.ds(o, _IBLK)], sem_xy)
            pltpu.async_copy(fuv_hbm.at[idxuv_v.at[pl.ds(o, _IBLK)]],
                             bufuv_v.at[pl.ds(o, _IBLK)], sem_uv)

    def blend_chunk(k):
        par = lax.rem(k, 2)
        pr = par * _RB
        pltpu.make_async_copy(fxy_hbm.at[pl.ds(0, _RB)],
                              bufxy_v.at[pl.ds(0, _RB)], sem_xy).wait()
        pltpu.make_async_copy(fuv_hbm.at[pl.ds(0, _RB)],
                              bufuv_v.at[pl.ds(0, _RB)], sem_uv).wait()

        @pl.loop(0, _NG)
        def blend(gi):
            off = gi * _L
            pts = pr + off + iota
            orow = par * _OT + (gi >> 3)
            for (buf, wv, cbase) in ((bufxy_v, wxy_v, 0),
                                     (bufuv_v, wuv_v, _LXY)):
                w11 = wv[pl.ds(pr + 0 * _C + off, _L)]
                w21 = wv[pl.ds(pr + 1 * _C + off, _L)]
                w12 = wv[pl.ds(pr + 2 * _C + off, _L)]
                w22 = wv[pl.ds(pr + 3 * _C + off, _L)]
                for l in range(_LXY):
                    col = jnp.full((_L,), l, jnp.int32)
                    g11 = plsc.load_gather(buf, [pts + 0 * _C, col])
                    g21 = plsc.load_gather(buf, [pts + 1 * _C, col])
                    g12 = plsc.load_gather(buf, [pts + 2 * _C, col])
                    g22 = plsc.load_gather(buf, [pts + 3 * _C, col])
                    acc = g11 * w11 + g21 * w21 + g12 * w12 + g22 * w22
                    out_v[orow, cbase + l, pl.ds((gi & 7) * _L, _L)] = acc

        cb = base_w + k * _C
        o0 = pl.multiple_of(cb // 128, _OT)
        pltpu.async_copy(out_v.at[pl.ds(par * _OT, _OT)],
                         out_hbm.at[pl.ds(o0, _OT)], sem_out)

    def wait_out():
        pltpu.make_async_copy(out_v.at[pl.ds(0, _OT)],
                              out_hbm.at[pl.ds(0, _OT)], sem_out).wait()

    fire_coords(0)

    @pl.loop(0, _NCH)
    def chunk(k):
        wait_coords()
        pass1_and_fire(k)

        @pl.when(k + 1 < _NCH)
        def _():
            fire_coords(k + 1)

        @pl.when(k > 1)
        def _():
            wait_out()

        @pl.when(k > 0)
        def _():
            blend_chunk(k - 1)

    blend_chunk(_NCH - 1)
    wait_out()
    wait_out()


_both_kernel = pl.kernel(
    _both_body,
    out_type=jax.ShapeDtypeStruct((_N // 128, 16, 128), jnp.float32),
    mesh=plsc.VectorSubcoreMesh(core_axis_name="c", subcore_axis_name="s"),
    compiler_params=pltpu.CompilerParams(needs_layout_passes=False,
                                         use_tc_tiling_on_sc=False),
    scratch_types=[
        pltpu.VMEM((2 * _C,), jnp.int32),
        pltpu.VMEM((2 * (_C // 128), 2, 128), jnp.float32),
        pltpu.VMEM((2 * _C,), jnp.float32),
        pltpu.VMEM((2 * _C,), jnp.float32),
        pltpu.VMEM((2 * _RB,), jnp.int32),
        pltpu.VMEM((2 * _RB,), jnp.int32),
        pltpu.VMEM((2 * _RB,), jnp.float32),
        pltpu.VMEM((2 * _RB,), jnp.float32),
        pltpu.VMEM((2 * _RB, _LXY), jnp.float32),
        pltpu.VMEM((2 * _RB, _LUV), jnp.float32),
        pltpu.VMEM((2 * (_C // 128), 16, 128), jnp.float32),
        pltpu.SemaphoreType.DMA,
        pltpu.SemaphoreType.DMA,
        pltpu.SemaphoreType.DMA,
        pltpu.SemaphoreType.DMA,
    ],
)


_TT = _M * _HX * (_HY // 128)
_TPW = _TT // _NW
_TB = 8
_TSTEPS = _TPW // _TB


def _tr_body(tin_hbm, tout_hbm, tin0, tin1, tout_v, sem_i, sem_o):
    wid = lax.axis_index("s") * _NC + lax.axis_index("c")
    tbase = wid * _TPW
    iota = lax.iota(jnp.int32, _L)
    d1 = iota & 7
    d2base = iota >> 3

    pltpu.async_copy(tin_hbm.at[pl.ds(tbase, _TB)], tin0, sem_i)

    @pl.loop(0, _TSTEPS)
    def step(c):
        tb = tbase + c * _TB

        @pl.when(c + 1 < _TSTEPS)
        def _():
            @pl.when(lax.rem(c, 2) == 0)
            def _():
                pltpu.async_copy(tin_hbm.at[pl.ds(tb + _TB, _TB)], tin1,
                                 sem_i)

            @pl.when(lax.rem(c, 2) == 1)
            def _():
                pltpu.async_copy(tin_hbm.at[pl.ds(tb + _TB, _TB)], tin0,
                                 sem_i)

        pltpu.make_async_copy(tin_hbm.at[pl.ds(0, _TB)], tin0, sem_i).wait()

        @pl.when(c > 0)
        def _():
            pltpu.make_async_copy(tout_v, tout_hbm.at[pl.ds(0, _TB)],
                                  sem_o).wait()

        for par in range(2):
            tin = (tin0, tin1)[par]

            @pl.when(lax.rem(c, 2) == par)
            def _():
                for t in range(_TB):
                    d0 = jnp.full((_L,), t, jnp.int32)
                    for g in range(64):
                        vals = plsc.load_gather(tin, [d0, d1, d2base + 2 * g])
                        tout_v[t, g, :] = vals

        pltpu.async_copy(tout_v, tout_hbm.at[pl.ds(tb, _TB)], sem_o)

    pltpu.make_async_copy(tout_v, tout_hbm.at[pl.ds(0, _TB)], sem_o).wait()


_xy_transpose = pl.kernel(
    _tr_body,
    out_type=jax.ShapeDtypeStruct((_TT, 64, 16), jnp.float32),
    mesh=plsc.VectorSubcoreMesh(core_axis_name="c", subcore_axis_name="s"),
    compiler_params=pltpu.CompilerParams(needs_layout_passes=False,
                                         use_tc_tiling_on_sc=False),
    scratch_types=[
        pltpu.VMEM((_TB, 8, 128), jnp.float32),
        pltpu.VMEM((_TB, 8, 128), jnp.float32),
        pltpu.VMEM((_TB, 64, 16), jnp.float32),
        pltpu.SemaphoreType.DMA,
        pltpu.SemaphoreType.DMA,
    ],
)


@jax.jit
def kernel(m, h, u, v, Fxy, Fuv):
    fxy3 = (Fxy.transpose(0, 1, 3, 2)
            .reshape(_M, _HX, _LXY, _HY // 128, 128)
            .transpose(0, 1, 3, 2, 4)
            .reshape(_TT, 8, 128))
    fxy = _xy_transpose(fxy3).reshape(_M * _HX * _HY, _LXY)
    fuv = Fuv.reshape(_M * _U * _V, _LUV)
    h3 = h.reshape(_N // 128, 128, 2).transpose(0, 2, 1)
    out = _both_kernel(m, h3, u, v, fxy, fuv)
    return out.transpose(0, 2, 1).reshape(_N, _LXY + _LUV)


# speedup vs baseline: 3.5897x; 1.0002x over previous
import jax
import jax.numpy as jnp
from jax import lax
from jax.experimental import pallas as pl
from jax.experimental.pallas import tpu as pltpu
from jax.experimental.pallas import tpu_sc as plsc

_M, _HX, _HY, _LXY = 8, 512, 512, 8
_U, _V, _LUV = 400, 400, 8
_N = 1048576

_NC, _NS, _L = 2, 16, 16
_NW = _NC * _NS
_PW = _N // _NW
_C = 512
_NCH = _PW // _C
_NG = _C // _L
_RB = 4 * _C
_IBLK = 128
_NBLK = _RB // _IBLK


def _corners(find, size):
    find = jnp.where(find >= float(size), jnp.full((_L,), float(size - 1)),
                     find)
    i1 = find.astype(jnp.int32)
    fr = find - i1.astype(jnp.float32)
    i2 = i1 + 1
    i2 = jnp.where(i2 >= size, jnp.zeros((_L,), jnp.int32), i2)
    return i1, i2, fr


def _plane_kernel(I, J, LD, use_h):
    def body(*refs):
        if use_h:
            (m_hbm, h_hbm, f_hbm, out_hbm,
             m_v, h_v, idx_v, w_v, buf_v, out_v,
             sem_in, sem_g, sem_out) = refs
        else:
            (m_hbm, u_hbm, v_hbm, f_hbm, out_hbm,
             m_v, u_v, v_v, idx_v, w_v, buf_v, out_v,
             sem_in, sem_g, sem_out) = refs
        wid = lax.axis_index("s") * _NC + lax.axis_index("c")
        base_w = wid * _PW
        iota = lax.iota(jnp.int32, _L)

        @pl.loop(0, _NCH)
        def chunk(k):
            cb = base_w + k * _C

            cm = pltpu.async_copy(m_hbm.at[pl.ds(cb, _C)], m_v, sem_in)
            if use_h:
                blk0 = pl.multiple_of(cb // 128, _C // 128)
                ca = pltpu.async_copy(h_hbm.at[pl.ds(blk0, _C // 128)], h_v,
                                      sem_in)
                cm.wait(); ca.wait()
            else:
                ca = pltpu.async_copy(u_hbm.at[pl.ds(cb, _C)], u_v, sem_in)
                cc = pltpu.async_copy(v_hbm.at[pl.ds(cb, _C)], v_v, sem_in)
                cm.wait(); ca.wait(); cc.wait()

            @pl.loop(0, _NG)
            def grp(gi):
                off = gi * _L
                mv = m_v[pl.ds(off, _L)]
                if use_h:
                    ci = h_v[gi >> 3, 0, pl.ds((gi & 7) * _L, _L)]
                    cj = h_v[gi >> 3, 1, pl.ds((gi & 7) * _L, _L)]
                    fi = (ci + 1.0) * (0.5 * I)
                    fj = (cj + 1.0) * (0.5 * J)
                else:
                    fi = u_v[pl.ds(off, _L)] * float(I)
                    fj = v_v[pl.ds(off, _L)] * float(J)
                i1, i2, ir = _corners(fi, I)
                j1, j2, jr = _corners(fj, J)
                base = mv * (I * J)
                a1 = base + i1 * J
                a2 = base + i2 * J
                idx_v[pl.ds(0 * _C + off, _L)] = a1 + j1
                idx_v[pl.ds(1 * _C + off, _L)] = a2 + j1
                idx_v[pl.ds(2 * _C + off, _L)] = a1 + j2
                idx_v[pl.ds(3 * _C + off, _L)] = a2 + j2
                omi = 1.0 - ir
                omj = 1.0 - jr
                w_v[pl.ds(0 * _C + off, _L)] = omi * omj
                w_v[pl.ds(1 * _C + off, _L)] = ir * omj
                w_v[pl.ds(2 * _C + off, _L)] = omi * jr
                w_v[pl.ds(3 * _C + off, _L)] = ir * jr

            @pl.loop(0, _NBLK)
            def fire(b):
                o = b * _IBLK
                pltpu.async_copy(f_hbm.at[idx_v.at[pl.ds(o, _IBLK)]],
                                 buf_v.at[pl.ds(o, _IBLK)], sem_g)

            @pl.when(k > 0)
            def _():
                pltpu.make_async_copy(out_v,
                                      out_hbm.at[pl.ds(0, _C // 128)],
                                      sem_out).wait()

            pltpu.make_async_copy(f_hbm.at[pl.ds(0, _RB)], buf_v,
                                  sem_g).wait()

            @pl.loop(0, _NG)
            def blend(gi):
                off = gi * _L
                pts = off + iota
                w11 = w_v[pl.ds(0 * _C + off, _L)]
                w21 = w_v[pl.ds(1 * _C + off, _L)]
                w12 = w_v[pl.ds(2 * _C + off, _L)]
                w22 = w_v[pl.ds(3 * _C + off, _L)]
                r11 = pts
                r21 = pts + 1 * _C
                r12 = pts + 2 * _C
                r22 = pts + 3 * _C
                for l in range(LD):
                    col = jnp.full((_L,), l, jnp.int32)
                    g11 = plsc.load_gather(buf_v, [r11, col])
                    g21 = plsc.load_gather(buf_v, [r21, col])
                    g12 = plsc.load_gather(buf_v, [r12, col])
                    g22 = plsc.load_gather(buf_v, [r22, col])
                    acc = g11 * w11 + g21 * w21 + g12 * w12 + g22 * w22
                    out_v[gi >> 3, l, pl.ds((gi & 7) * _L, _L)] = acc

            o0 = pl.multiple_of(cb // 128, _C // 128)
            pltpu.async_copy(out_v, out_hbm.at[pl.ds(o0, _C // 128)],
                             sem_out)

        pltpu.make_async_copy(out_v, out_hbm.at[pl.ds(0, _C // 128)],
                              sem_out).wait()

    coord_scratch = ([pltpu.VMEM((_C // 128, 2, 128), jnp.float32)]
                     if use_h else
                     [pltpu.VMEM((_C,), jnp.float32),
                      pltpu.VMEM((_C,), jnp.float32)])
    return pl.kernel(
        body,
        out_type=jax.ShapeDtypeStruct((_N // 128, LD, 128), jnp.float32),
        mesh=plsc.VectorSubcoreMesh(core_axis_name="c",
                                    subcore_axis_name="s"),
        compiler_params=pltpu.CompilerParams(needs_layout_passes=False,
                                             use_tc_tiling_on_sc=False),
        scratch_types=[pltpu.VMEM((_C,), jnp.int32)] + coord_scratch + [
            pltpu.VMEM((_RB,), jnp.int32),
            pltpu.VMEM((_RB,), jnp.float32),
            pltpu.VMEM((_RB, LD), jnp.float32),
            pltpu.VMEM((_C // 128, LD, 128), jnp.float32),
            pltpu.SemaphoreType.DMA,
            pltpu.SemaphoreType.DMA,
            pltpu.SemaphoreType.DMA,
        ],
    )


_xy_kernel = _plane_kernel(_HX, _HY, _LXY, use_h=True)
_uv_kernel = _plane_kernel(_U, _V, _LUV, use_h=False)


def _both_body(m_hbm, h_hbm, u_hbm, v_hbm, fxy_hbm, fuv_hbm, out_hbm,
               m_v, h_v, u_v, v_v, idxxy_v, idxuv_v, wxy_v, wuv_v,
               bufxy_v, bufuv_v, out_v, sem_in, sem_xy, sem_uv, sem_out):
    wid = lax.axis_index("s") * _NC + lax.axis_index("c")
    base_w = wid * _PW
    iota = lax.iota(jnp.int32, _L)
    _OT = _C // 128

    def fire_coords(k):
        cb = base_w + k * _C
        par = lax.rem(k, 2)
        blk0 = pl.multiple_of(cb // 128, _OT)
        pltpu.async_copy(m_hbm.at[pl.ds(cb, _C)],
                         m_v.at[pl.ds(par * _C, _C)], sem_in)
        pltpu.async_copy(h_hbm.at[pl.ds(blk0, _OT)],
                         h_v.at[pl.ds(par * _OT, _OT)], sem_in)
        pltpu.async_copy(u_hbm.at[pl.ds(cb, _C)],
                         u_v.at[pl.ds(par * _C, _C)], sem_in)
        pltpu.async_copy(v_hbm.at[pl.ds(cb, _C)],
                         v_v.at[pl.ds(par * _C, _C)], sem_in)

    def wait_coords():
        pltpu.make_async_copy(m_hbm.at[pl.ds(0, _C)],
                              m_v.at[pl.ds(0, _C)], sem_in).wait()
        pltpu.make_async_copy(h_hbm.at[pl.ds(0, _OT)],
                              h_v.at[pl.ds(0, _OT)], sem_in).wait()
        pltpu.make_async_copy(u_hbm.at[pl.ds(0, _C)],
                              u_v.at[pl.ds(0, _C)], sem_in).wait()
        pltpu.make_async_copy(v_hbm.at[pl.ds(0, _C)],
                              v_v.at[pl.ds(0, _C)], sem_in).wait()

    def pass1_and_fire(k):
        par = lax.rem(k, 2)
        pc = par * _C
        pr = par * _RB

        @pl.loop(0, _NG)
        def grp(gi):
            off = gi * _L
            mv = m_v[pl.ds(pc + off, _L)]
            hrow = par * (_C // 128) + (gi >> 3)
            ci = h_v[hrow, 0, pl.ds((gi & 7) * _L, _L)]
            cj = h_v[hrow, 1, pl.ds((gi & 7) * _L, _L)]
            i1, i2, ir = _corners((ci + 1.0) * (0.5 * _HX), _HX)
            j1, j2, jr = _corners((cj + 1.0) * (0.5 * _HY), _HY)
            base = mv * (_HX * _HY)
            a1 = base + i1 * _HY
            a2 = base + i2 * _HY
            idxxy_v[pl.ds(pr + 0 * _C + off, _L)] = a1 + j1
            idxxy_v[pl.ds(pr + 1 * _C + off, _L)] = a2 + j1
            idxxy_v[pl.ds(pr + 2 * _C + off, _L)] = a1 + j2
            idxxy_v[pl.ds(pr + 3 * _C + off, _L)] = a2 + j2
            omi = 1.0 - ir
            omj = 1.0 - jr
            wxy_v[pl.ds(pr + 0 * _C + off, _L)] = omi * omj
            wxy_v[pl.ds(pr + 1 * _C + off, _L)] = ir * omj
            wxy_v[pl.ds(pr + 2 * _C + off, _L)] = omi * jr
            wxy_v[pl.ds(pr + 3 * _C + off, _L)] = ir * jr

            p1, p2, prf = _corners(u_v[pl.ds(pc + off, _L)] * float(_U), _U)
            q1, q2, qrf = _corners(v_v[pl.ds(pc + off, _L)] * float(_V), _V)
            baseu = mv * (_U * _V)
            b1 = baseu + p1 * _V
            b2 = baseu + p2 * _V
            idxuv_v[pl.ds(pr + 0 * _C + off, _L)] = b1 + q1
            idxuv_v[pl.ds(pr + 1 * _C + off, _L)] = b2 + q1
            idxuv_v[pl.ds(pr + 2 * _C + off, _L)] = b1 + q2
            idxuv_v[pl.ds(pr + 3 * _C + off, _L)] = b2 + q2
            omp = 1.0 - prf
            omq = 1.0 - qrf
            wuv_v[pl.ds(pr + 0 * _C + off, _L)] = omp * omq
            wuv_v[pl.ds(pr + 1 * _C + off, _L)] = prf * omq
            wuv_v[pl.ds(pr + 2 * _C + off, _L)] = omp * qrf
            wuv_v[pl.ds(pr + 3 * _C + off, _L)] = prf * qrf

        @pl.loop(0, _NBLK)
        def fire(b):
            o = pr + b * _IBLK
            pltpu.async_copy(fxy_hbm.at[idxxy_v.at[pl.ds(o, _IBLK)]],
                             bufxy_v.at[pl.ds(o, _IBLK)], sem_xy)
            pltpu.async_copy(fuv_hbm.at[idxuv_v.at[pl.ds(o, _IBLK)]],
                             bufuv_v.at[pl.ds(o, _IBLK)], sem_uv)

    def blend_chunk(k):
        par = lax.rem(k, 2)
        pr = par * _RB
        pltpu.make_async_copy(fxy_hbm.at[pl.ds(0, _RB)],
                              bufxy_v.at[pl.ds(0, _RB)], sem_xy).wait()
        pltpu.make_async_copy(fuv_hbm.at[pl.ds(0, _RB)],
                              bufuv_v.at[pl.ds(0, _RB)], sem_uv).wait()

        @pl.loop(0, _NG)
        def blend(gi):
            off = gi * _L
            pts = pr + off + iota
            orow = par * _OT + (gi >> 3)
            for (buf, wv, cbase) in ((bufxy_v, wxy_v, 0),
                                     (bufuv_v, wuv_v, _LXY)):
                w11 = wv[pl.ds(pr + 0 * _C + off, _L)]
                w21 = wv[pl.ds(pr + 1 * _C + off, _L)]
                w12 = wv[pl.ds(pr + 2 * _C + off, _L)]
                w22 = wv[pl.ds(pr + 3 * _C + off, _L)]
                for l in range(_LXY):
                    col = jnp.full((_L,), l, jnp.int32)
                    g11 = plsc.load_gather(buf, [pts + 0 * _C, col])
                    g21 = plsc.load_gather(buf, [pts + 1 * _C, col])
                    g12 = plsc.load_gather(buf, [pts + 2 * _C, col])
                    g22 = plsc.load_gather(buf, [pts + 3 * _C, col])
                    acc = g11 * w11 + g21 * w21 + g12 * w12 + g22 * w22
                    out_v[orow, cbase + l, pl.ds((gi & 7) * _L, _L)] = acc

        cb = base_w + k * _C
        o0 = pl.multiple_of(cb // 128, _OT)
        pltpu.async_copy(out_v.at[pl.ds(par * _OT, _OT)],
                         out_hbm.at[pl.ds(o0, _OT)], sem_out)

    def wait_out():
        pltpu.make_async_copy(out_v.at[pl.ds(0, _OT)],
                              out_hbm.at[pl.ds(0, _OT)], sem_out).wait()

    fire_coords(0)

    @pl.loop(0, _NCH)
    def chunk(k):
        wait_coords()
        pass1_and_fire(k)

        @pl.when(k + 1 < _NCH)
        def _():
            fire_coords(k + 1)

        @pl.when(k > 1)
        def _():
            wait_out()

        @pl.when(k > 0)
        def _():
            blend_chunk(k - 1)

    blend_chunk(_NCH - 1)
    wait_out()
    wait_out()


_both_kernel = pl.kernel(
    _both_body,
    out_type=jax.ShapeDtypeStruct((_N // 128, 16, 128), jnp.float32),
    mesh=plsc.VectorSubcoreMesh(core_axis_name="c", subcore_axis_name="s"),
    compiler_params=pltpu.CompilerParams(needs_layout_passes=False,
                                         use_tc_tiling_on_sc=False),
    scratch_types=[
        pltpu.VMEM((2 * _C,), jnp.int32),
        pltpu.VMEM((2 * (_C // 128), 2, 128), jnp.float32),
        pltpu.VMEM((2 * _C,), jnp.float32),
        pltpu.VMEM((2 * _C,), jnp.float32),
        pltpu.VMEM((2 * _RB,), jnp.int32),
        pltpu.VMEM((2 * _RB,), jnp.int32),
        pltpu.VMEM((2 * _RB,), jnp.float32),
        pltpu.VMEM((2 * _RB,), jnp.float32),
        pltpu.VMEM((2 * _RB, _LXY), jnp.float32),
        pltpu.VMEM((2 * _RB, _LUV), jnp.float32),
        pltpu.VMEM((2 * (_C // 128), 16, 128), jnp.float32),
        pltpu.SemaphoreType.DMA,
        pltpu.SemaphoreType.DMA,
        pltpu.SemaphoreType.DMA,
        pltpu.SemaphoreType.DMA,
    ],
)


_TT = _M * _HX * (_HY // 128)
_TPW = _TT // _NW
_TB = 8
_TSTEPS = _TPW // _TB


def _tr_body(tin_hbm, tout_hbm, tin0, tin1, tout_v, sem_i, sem_o):
    wid = lax.axis_index("s") * _NC + lax.axis_index("c")
    tbase = wid * _TPW
    iota = lax.iota(jnp.int32, _L)
    d1 = iota & 7
    d2base = iota >> 3

    pltpu.async_copy(tin_hbm.at[pl.ds(tbase, _TB)], tin0, sem_i)

    @pl.loop(0, _TSTEPS)
    def step(c):
        tb = tbase + c * _TB

        @pl.when(c + 1 < _TSTEPS)
        def _():
            @pl.when(lax.rem(c, 2) == 0)
            def _():
                pltpu.async_copy(tin_hbm.at[pl.ds(tb + _TB, _TB)], tin1,
                                 sem_i)

            @pl.when(lax.rem(c, 2) == 1)
            def _():
                pltpu.async_copy(tin_hbm.at[pl.ds(tb + _TB, _TB)], tin0,
                                 sem_i)

        pltpu.make_async_copy(tin_hbm.at[pl.ds(0, _TB)], tin0, sem_i).wait()

        @pl.when(c > 0)
        def _():
            pltpu.make_async_copy(tout_v, tout_hbm.at[pl.ds(0, _TB)],
                                  sem_o).wait()

        for par in range(2):
            tin = (tin0, tin1)[par]

            @pl.when(lax.rem(c, 2) == par)
            def _():
                for t in range(_TB):
                    d0 = jnp.full((_L,), t, jnp.int32)
                    for g in range(64):
                        vals = plsc.load_gather(tin, [d0, d1, d2base + 2 * g])
                        tout_v[t, g >> 3, pl.ds((g & 7) * _L, _L)] = vals

        pltpu.async_copy(tout_v, tout_hbm.at[pl.ds(tb, _TB)], sem_o)

    pltpu.make_async_copy(tout_v, tout_hbm.at[pl.ds(0, _TB)], sem_o).wait()


_xy_transpose = pl.kernel(
    _tr_body,
    out_type=jax.ShapeDtypeStruct((_TT, 8, 128), jnp.float32),
    mesh=plsc.VectorSubcoreMesh(core_axis_name="c", subcore_axis_name="s"),
    compiler_params=pltpu.CompilerParams(needs_layout_passes=False,
                                         use_tc_tiling_on_sc=True),
    scratch_types=[
        pltpu.VMEM((_TB, 8, 128), jnp.float32),
        pltpu.VMEM((_TB, 8, 128), jnp.float32),
        pltpu.VMEM((_TB, 8, 128), jnp.float32),
        pltpu.SemaphoreType.DMA,
        pltpu.SemaphoreType.DMA,
    ],
)


@jax.jit
def kernel(m, h, u, v, Fxy, Fuv):
    fxy3 = (Fxy.transpose(0, 1, 3, 2)
            .reshape(_M, _HX, _LXY, _HY // 128, 128)
            .transpose(0, 1, 3, 2, 4)
            .reshape(_TT, 8, 128))
    fxy = _xy_transpose(fxy3).reshape(_M * _HX * _HY, _LXY)
    fuv = Fuv.reshape(_M * _U * _V, _LUV)
    h3 = h.reshape(_N // 128, 128, 2).transpose(0, 2, 1)
    out = _both_kernel(m, h3, u, v, fxy, fuv)
    return out.transpose(0, 2, 1).reshape(_N, _LXY + _LUV)
